# Initial kernel scaffold; baseline (speedup 1.0000x reference)
import jax
import jax.numpy as jnp
from jax.experimental import pallas as pl


def kernel(inputs, luts_float):
    raise NotImplementedError("write your pallas kernel here")


# speedup vs baseline: 24.6744x; 24.6744x over previous
import functools

import jax
import jax.numpy as jnp
from jax import lax
from jax.experimental import pallas as pl
from jax.experimental.pallas import tpu as pltpu
from jax.experimental.pallas import tpu_sc as plsc

L = 16
NC = 2
NS = 16
NW = NC * NS


def kernel(inputs, luts_float):
    B, NI = inputs.shape
    NI2, LS, D = luts_float.shape
    assert NI2 == NI and B % NW == 0 and D % L == 0
    bpw = B // NW
    nv = (NI + L - 1) // L
    NIP = nv * L
    NG = ((NI + 7) // 8) * 8
    dv = D // L
    off = float(LS) / 2.0
    hi = float(LS) - 1.0001

    table = luts_float.reshape(NI * LS, D)
    xpad = jnp.pad(inputs, ((0, 0), (0, NIP - NI)))

    mesh = plsc.VectorSubcoreMesh(
        core_axis_name="c", subcore_axis_name="s", num_cores=NC, num_subcores=NS
    )

    @functools.partial(
        pl.kernel,
        out_type=jax.ShapeDtypeStruct((B, D), jnp.float32),
        mesh=mesh,
        scratch_types=[
            pltpu.VMEM((bpw, NIP), jnp.float32),
            pltpu.VMEM((bpw, NIP), jnp.int32),
            pltpu.VMEM((bpw, NIP), jnp.int32),
            pltpu.VMEM((bpw, NIP), jnp.float32),
            pltpu.VMEM((NG, D), jnp.float32),
            pltpu.VMEM((NG, D), jnp.float32),
            pltpu.VMEM((bpw, D), jnp.float32),
            pltpu.SemaphoreType.DMA,
            pltpu.SemaphoreType.DMA,
        ],
    )
    def lut_kernel(x_hbm, tab_hbm, out_hbm, xin, idx0, idx1, frac,
                   rows0, rows1, accb, sem0, sem1):
        wid = lax.axis_index("s") * NC + lax.axis_index("c")
        base = wid * bpw
        pltpu.sync_copy(x_hbm.at[pl.ds(base, bpw)], xin)

        def prep_row(b, carry):
            for v in range(nv):
                xv = xin[b, pl.ds(v * L, L)]
                x = jnp.minimum(jnp.maximum(xv + off, 0.0), hi)
                x0 = x.astype(jnp.int32)
                fr = x - x0.astype(jnp.float32)
                fl = x0 + (lax.iota(jnp.int32, L) + v * L) * LS
                if (v + 1) * L > NI:
                    ok = (lax.iota(jnp.int32, L) + v * L) < NI
                    fl = jnp.where(ok, fl, 0)
                idx0[b, pl.ds(v * L, L)] = fl
                idx1[b, pl.ds(v * L, L)] = fl + 1
                frac[b, pl.ds(v * L, L)] = fr
            return carry

        lax.fori_loop(0, bpw, prep_row, 0)

        nv_full = NI // L
        tail = NI - nv_full * L

        def accum_feature(i, fscalar, accs):
            fv = jnp.full((L,), fscalar, jnp.float32)
            new = []
            for j in range(dv):
                r0 = rows0[i, pl.ds(j * L, L)]
                r1 = rows1[i, pl.ds(j * L, L)]
                new.append(accs[j] + (r0 + fv * (r1 - r0)))
            return tuple(new)

        def do_row(b, carry):
            c0 = pltpu.async_copy(tab_hbm.at[idx0.at[b, pl.ds(0, NG)]], rows0, sem0)
            c1 = pltpu.async_copy(tab_hbm.at[idx1.at[b, pl.ds(0, NG)]], rows1, sem1)
            c0.wait()
            c1.wait()

            def group(v, accs):
                fvec = frac[b, pl.ds(v * L, L)]
                for l in range(L):
                    accs = accum_feature(v * L + l, fvec[l], accs)
                return accs

            accs = lax.fori_loop(
                0, nv_full, group,
                tuple(jnp.zeros((L,), jnp.float32) for _ in range(dv)),
            )
            if tail:
                fvec = frac[b, pl.ds(nv_full * L, L)]
                for l in range(tail):
                    accs = accum_feature(nv_full * L + l, fvec[l], accs)
            for j in range(dv):
                accb[b, pl.ds(j * L, L)] = accs[j]
            return carry

        lax.fori_loop(0, bpw, do_row, 0)
        pltpu.sync_copy(accb, out_hbm.at[pl.ds(base, bpw)])

    return lut_kernel(xpad, table)
